# baseline (speedup 1.0000x reference)
import jax
import jax.numpy as jnp
from jax.experimental import pallas as pl
from jax.experimental.pallas import tpu as pltpu

_N, _K, _D, _L = 2048, 48, 128, 3
_BN = 128


def _gelu(x):
    return 0.5 * x * (1.0 + jax.lax.erf(x * 0.7071067811865476))


def _ln(x, w, b, eps=1e-5):
    mu = jnp.mean(x, axis=-1, keepdims=True)
    xc = x - mu
    var = jnp.mean(xc * xc, axis=-1, keepdims=True)
    return xc * jax.lax.rsqrt(var + eps) * w + b


def _decoder_kernel(nf_ref, edge_ref, mask_ref,
                    w0h_ref, w0n_ref, w0e_ref, b0_ref,
                    w1_ref, b1_ref, w2s_ref, b2s_ref,
                    ln1w_ref, ln1b_ref,
                    dw0_ref, db0_ref, dw1_ref, db1_ref,
                    ln2w_ref, ln2b_ref,
                    out_ref):
    nf = nf_ref[...]
    edge2 = edge_ref[...].reshape(_BN * _K, _D)
    h = nf
    for l in range(_L):
        t0 = jnp.dot(h, w0h_ref[l], preferred_element_type=jnp.float32)
        t0 = t0 + jnp.dot(nf, w0n_ref[l], preferred_element_type=jnp.float32)
        t0 = t0 + b0_ref[l]
        e0 = jnp.dot(edge2, w0e_ref[l], preferred_element_type=jnp.float32)
        x1 = _gelu(e0.reshape(_BN, _K, _D) + t0[:, None, :]).reshape(_BN * _K, _D)
        x2 = _gelu(jnp.dot(x1.astype(jnp.bfloat16), w1_ref[l],
                           preferred_element_type=jnp.float32)
                   + b1_ref[l])
        s = jnp.sum(x2.reshape(_BN, _K, _D), axis=1)
        agg = jnp.dot(s, w2s_ref[l], preferred_element_type=jnp.float32) + b2s_ref[l]
        h = _ln(h + agg, ln1w_ref[l], ln1b_ref[l])
        d1 = _gelu(jnp.dot(h, dw0_ref[l], preferred_element_type=jnp.float32)
                   + db0_ref[l])
        d2 = jnp.dot(d1, dw1_ref[l], preferred_element_type=jnp.float32) + db1_ref[l]
        h = _ln(h + d2, ln2w_ref[l], ln2b_ref[l])
    out_ref[...] = h * mask_ref[...]


def kernel(node_features, edge_features, mask, m_w0, m_b0, m_w1, m_b1, m_w2,
           m_b2, ln1_w, ln1_b, d_w0, d_b0, d_w1, d_b1, ln2_w, ln2_b):
    tr = lambda w: jnp.transpose(w, (0, 2, 1))
    w0h = tr(m_w0[:, :, 0 * _D:1 * _D])
    w0n = tr(m_w0[:, :, 1 * _D:2 * _D])
    w0e = tr(m_w0[:, :, 3 * _D:4 * _D]).astype(jnp.bfloat16)
    w1 = tr(m_w1).astype(jnp.bfloat16)
    w2s = tr(m_w2) * (1.0 / 30.0)
    dw0 = tr(d_w0)
    dw1 = tr(d_w1)
    col = lambda b: b.reshape(_L, 1, _D)
    b0 = col(m_b0)
    b1 = col(m_b1)
    b2s = col(m_b2) * (_K / 30.0)
    mask2 = mask[:, None]
    edge_bf = edge_features.astype(jnp.bfloat16)

    full = lambda a: pl.BlockSpec(a.shape, lambda i: (0,) * a.ndim)
    weights = (w0h, w0n, w0e, b0, w1, b1, w2s, b2s,
               col(ln1_w), col(ln1_b), dw0, col(d_b0), dw1, col(d_b1),
               col(ln2_w), col(ln2_b))
    return pl.pallas_call(
        _decoder_kernel,
        grid=(_N // _BN,),
        in_specs=[
            pl.BlockSpec((_BN, _D), lambda i: (i, 0)),
            pl.BlockSpec((_BN, _K, _D), lambda i: (i, 0, 0)),
            pl.BlockSpec((_BN, 1), lambda i: (i, 0)),
        ] + [full(w) for w in weights],
        out_specs=pl.BlockSpec((_BN, _D), lambda i: (i, 0)),
        out_shape=jax.ShapeDtypeStruct((_N, _D), jnp.float32),
        compiler_params=pltpu.CompilerParams(
            dimension_semantics=("parallel",)),
    )(node_features, edge_bf, mask2, *weights)


# speedup vs baseline: 1.1519x; 1.1519x over previous
import jax
import jax.numpy as jnp
from jax.experimental import pallas as pl
from jax.experimental.pallas import tpu as pltpu

_N, _K, _D, _L = 2048, 48, 128, 3
_BN = 128


def _gelu(x):
    return 0.5 * x * (1.0 + jax.lax.erf(x * 0.7071067811865476))


def _ln(x, w, b, eps=1e-5):
    mu = jnp.mean(x, axis=-1, keepdims=True)
    xc = x - mu
    var = jnp.mean(xc * xc, axis=-1, keepdims=True)
    return xc * jax.lax.rsqrt(var + eps) * w + b


def _decoder_kernel(nf_ref, edge_ref, mask_ref,
                    w0h_ref, w0n_ref, w0e_ref, b0_ref,
                    w1_ref, b1_ref, w2s_ref, b2s_ref,
                    ln1w_ref, ln1b_ref,
                    dw0_ref, db0_ref, dw1_ref, db1_ref,
                    ln2w_ref, ln2b_ref,
                    out_ref):
    nf = nf_ref[...]
    edge2 = edge_ref[...].reshape(_BN * _K, _D).astype(jnp.bfloat16)
    h = nf
    for l in range(_L):
        t0 = jnp.dot(h, w0h_ref[l], preferred_element_type=jnp.float32)
        t0 = t0 + jnp.dot(nf, w0n_ref[l], preferred_element_type=jnp.float32)
        t0 = t0 + b0_ref[l]
        e0 = jnp.dot(edge2, w0e_ref[l], preferred_element_type=jnp.float32)
        x1 = _gelu(e0.reshape(_BN, _K, _D) + t0[:, None, :]).reshape(_BN * _K, _D)
        x2 = _gelu(jnp.dot(x1.astype(jnp.bfloat16), w1_ref[l],
                           preferred_element_type=jnp.float32)
                   + b1_ref[l])
        s = jnp.sum(x2.reshape(_BN, _K, _D), axis=1)
        agg = jnp.dot(s, w2s_ref[l], preferred_element_type=jnp.float32) + b2s_ref[l]
        h = _ln(h + agg, ln1w_ref[l], ln1b_ref[l])
        d1 = _gelu(jnp.dot(h, dw0_ref[l], preferred_element_type=jnp.float32)
                   + db0_ref[l])
        d2 = jnp.dot(d1, dw1_ref[l], preferred_element_type=jnp.float32) + db1_ref[l]
        h = _ln(h + d2, ln2w_ref[l], ln2b_ref[l])
    out_ref[...] = h * mask_ref[...]


def kernel(node_features, edge_features, mask, m_w0, m_b0, m_w1, m_b1, m_w2,
           m_b2, ln1_w, ln1_b, d_w0, d_b0, d_w1, d_b1, ln2_w, ln2_b):
    tr = lambda w: jnp.transpose(w, (0, 2, 1))
    w0h = tr(m_w0[:, :, 0 * _D:1 * _D])
    w0n = tr(m_w0[:, :, 1 * _D:2 * _D])
    w0e = tr(m_w0[:, :, 3 * _D:4 * _D]).astype(jnp.bfloat16)
    w1 = tr(m_w1).astype(jnp.bfloat16)
    w2s = tr(m_w2) * (1.0 / 30.0)
    dw0 = tr(d_w0)
    dw1 = tr(d_w1)
    col = lambda b: b.reshape(_L, 1, _D)
    b0 = col(m_b0)
    b1 = col(m_b1)
    b2s = col(m_b2) * (_K / 30.0)
    mask2 = mask[:, None]

    full = lambda a: pl.BlockSpec(a.shape, lambda i: (0,) * a.ndim)
    weights = (w0h, w0n, w0e, b0, w1, b1, w2s, b2s,
               col(ln1_w), col(ln1_b), dw0, col(d_b0), dw1, col(d_b1),
               col(ln2_w), col(ln2_b))
    return pl.pallas_call(
        _decoder_kernel,
        grid=(_N // _BN,),
        in_specs=[
            pl.BlockSpec((_BN, _D), lambda i: (i, 0)),
            pl.BlockSpec((_BN, _K, _D), lambda i: (i, 0, 0)),
            pl.BlockSpec((_BN, 1), lambda i: (i, 0)),
        ] + [full(w) for w in weights],
        out_specs=pl.BlockSpec((_BN, _D), lambda i: (i, 0)),
        out_shape=jax.ShapeDtypeStruct((_N, _D), jnp.float32),
        compiler_params=pltpu.CompilerParams(
            dimension_semantics=("parallel",)),
    )(node_features, edge_features, mask2, *weights)


# speedup vs baseline: 1.3585x; 1.1793x over previous
import jax
import jax.numpy as jnp
from jax.experimental import pallas as pl
from jax.experimental.pallas import tpu as pltpu

_N, _K, _D, _L = 2048, 48, 128, 3
_BN = 128


def _g(x):
    return x * (1.0 + jax.lax.erf(x))


def _ln(x, w, b, eps=1e-5):
    mu = jnp.mean(x, axis=-1, keepdims=True)
    xc = x - mu
    var = jnp.mean(xc * xc, axis=-1, keepdims=True)
    return xc * jax.lax.rsqrt(var + eps) * w + b


def _decoder_kernel(nf_ref, edge_ref, mask_ref,
                    w0h_ref, w0n_ref, w0e_ref, b0_ref,
                    w1_ref, b1_ref, w2s_ref, b2s_ref,
                    ln1w_ref, ln1b_ref,
                    dw0_ref, db0_ref, dw1_ref, db1_ref,
                    ln2w_ref, ln2b_ref,
                    out_ref):
    nf = nf_ref[...]
    edge2 = edge_ref[...].reshape(_BN * _K, _D)
    h = nf
    for l in range(_L):
        t0 = jnp.dot(h, w0h_ref[l], preferred_element_type=jnp.float32)
        t0 = t0 + jnp.dot(nf, w0n_ref[l], preferred_element_type=jnp.float32)
        t0 = t0 + b0_ref[l]
        e0 = jnp.dot(edge2, w0e_ref[l], preferred_element_type=jnp.float32)
        x1 = _g(e0.reshape(_BN, _K, _D) + t0[:, None, :]).reshape(_BN * _K, _D)
        x2 = _g(jnp.dot(x1, w1_ref[l], preferred_element_type=jnp.float32)
                + b1_ref[l])
        s = jnp.sum(x2.reshape(_BN, _K, _D), axis=1)
        agg = jnp.dot(s, w2s_ref[l], preferred_element_type=jnp.float32) + b2s_ref[l]
        h = _ln(h + agg, ln1w_ref[l], ln1b_ref[l])
        d1 = _g(jnp.dot(h, dw0_ref[l], preferred_element_type=jnp.float32)
                + db0_ref[l])
        d2 = jnp.dot(d1, dw1_ref[l], preferred_element_type=jnp.float32) \
            + db1_ref[l]
        h = _ln(h + d2, ln2w_ref[l], ln2b_ref[l])
    out_ref[...] = h * mask_ref[...]


def kernel(node_features, edge_features, mask, m_w0, m_b0, m_w1, m_b1, m_w2,
           m_b2, ln1_w, ln1_b, d_w0, d_b0, d_w1, d_b1, ln2_w, ln2_b):
    r = 0.7071067811865476
    tr = lambda w: jnp.transpose(w, (0, 2, 1))
    w0h = tr(m_w0[:, :, 0 * _D:1 * _D]) * r
    w0n = tr(m_w0[:, :, 1 * _D:2 * _D]) * r
    w0e = tr(m_w0[:, :, 3 * _D:4 * _D]) * r
    w1 = tr(m_w1) * 0.5
    w2s = tr(m_w2) * (r / 30.0)
    dw0 = tr(d_w0) * r
    dw1 = tr(d_w1) * r
    col = lambda b: b.reshape(_L, 1, _D)
    b0 = col(m_b0) * r
    b1 = col(m_b1) * r
    b2s = col(m_b2) * (_K / 30.0)
    mask2 = mask[:, None]

    full = lambda a: pl.BlockSpec(a.shape, lambda i: (0,) * a.ndim)
    weights = (w0h, w0n, w0e, b0, w1, b1, w2s, b2s,
               col(ln1_w), col(ln1_b), dw0, col(d_b0) * r, dw1, col(d_b1),
               col(ln2_w), col(ln2_b))
    return pl.pallas_call(
        _decoder_kernel,
        grid=(_N // _BN,),
        in_specs=[
            pl.BlockSpec((_BN, _D), lambda i: (i, 0)),
            pl.BlockSpec((_BN, _K, _D), lambda i: (i, 0, 0)),
            pl.BlockSpec((_BN, 1), lambda i: (i, 0)),
        ] + [full(w) for w in weights],
        out_specs=pl.BlockSpec((_BN, _D), lambda i: (i, 0)),
        out_shape=jax.ShapeDtypeStruct((_N, _D), jnp.float32),
        compiler_params=pltpu.CompilerParams(
            dimension_semantics=("parallel",)),
    )(node_features, edge_features, mask2, *weights)


# speedup vs baseline: 1.6148x; 1.1887x over previous
import jax
import jax.numpy as jnp
from jax.experimental import pallas as pl
from jax.experimental.pallas import tpu as pltpu

_N, _K, _D, _L = 2048, 48, 128, 3
_BN = 256


def _g(x):
    return x * (1.0 + jax.lax.erf(x))


def _ln(x, w, b, eps=1e-5):
    mu = jnp.mean(x, axis=-1, keepdims=True)
    xc = x - mu
    var = jnp.mean(xc * xc, axis=-1, keepdims=True)
    return xc * jax.lax.rsqrt(var + eps) * w + b


def _decoder_kernel(nf_ref, edge_ref, mask_ref,
                    w0h_ref, w0n_ref, w0e_ref, b0_ref,
                    w1_ref, b1_ref, w2s_ref, b2s_ref,
                    ln1w_ref, ln1b_ref,
                    dw0_ref, db0_ref, dw1_ref, db1_ref,
                    ln2w_ref, ln2b_ref,
                    out_ref):
    nf = nf_ref[...]
    edge2 = edge_ref[...].reshape(_BN * _K, _D)
    h = nf
    e0 = jnp.dot(edge2, w0e_ref[0], preferred_element_type=jnp.float32)
    for l in range(_L):
        t0 = jnp.dot(h, w0h_ref[l], preferred_element_type=jnp.float32)
        t0 = t0 + jnp.dot(nf, w0n_ref[l], preferred_element_type=jnp.float32)
        t0 = t0 + b0_ref[l]
        x1 = _g(e0.reshape(_BN, _K, _D) + t0[:, None, :]).reshape(_BN * _K, _D)
        x2 = _g(jnp.dot(x1, w1_ref[l], preferred_element_type=jnp.float32)
                + b1_ref[l])
        if l + 1 < _L:
            e0 = jnp.dot(edge2, w0e_ref[l + 1],
                         preferred_element_type=jnp.float32)
        s = jnp.sum(x2.reshape(_BN, _K, _D), axis=1)
        agg = jnp.dot(s, w2s_ref[l], preferred_element_type=jnp.float32) + b2s_ref[l]
        h = _ln(h + agg, ln1w_ref[l], ln1b_ref[l])
        d1 = _g(jnp.dot(h, dw0_ref[l], preferred_element_type=jnp.float32)
                + db0_ref[l])
        d2 = jnp.dot(d1, dw1_ref[l], preferred_element_type=jnp.float32) \
            + db1_ref[l]
        h = _ln(h + d2, ln2w_ref[l], ln2b_ref[l])
    out_ref[...] = h * mask_ref[...]


def kernel(node_features, edge_features, mask, m_w0, m_b0, m_w1, m_b1, m_w2,
           m_b2, ln1_w, ln1_b, d_w0, d_b0, d_w1, d_b1, ln2_w, ln2_b):
    r = 0.7071067811865476
    tr = lambda w: jnp.transpose(w, (0, 2, 1))
    w0h = tr(m_w0[:, :, 0 * _D:1 * _D]) * r
    w0n = tr(m_w0[:, :, 1 * _D:2 * _D]) * r
    w0e = tr(m_w0[:, :, 3 * _D:4 * _D]) * r
    w1 = tr(m_w1) * 0.5
    w2s = tr(m_w2) * (r / 30.0)
    dw0 = tr(d_w0) * r
    dw1 = tr(d_w1) * r
    col = lambda b: b.reshape(_L, 1, _D)
    b0 = col(m_b0) * r
    b1 = col(m_b1) * r
    b2s = col(m_b2) * (_K / 30.0)
    mask2 = mask[:, None]

    full = lambda a: pl.BlockSpec(a.shape, lambda i: (0,) * a.ndim)
    weights = (w0h, w0n, w0e, b0, w1, b1, w2s, b2s,
               col(ln1_w), col(ln1_b), dw0, col(d_b0) * r, dw1, col(d_b1),
               col(ln2_w), col(ln2_b))
    return pl.pallas_call(
        _decoder_kernel,
        grid=(_N // _BN,),
        in_specs=[
            pl.BlockSpec((_BN, _D), lambda i: (i, 0)),
            pl.BlockSpec((_BN, _K, _D), lambda i: (i, 0, 0)),
            pl.BlockSpec((_BN, 1), lambda i: (i, 0)),
        ] + [full(w) for w in weights],
        out_specs=pl.BlockSpec((_BN, _D), lambda i: (i, 0)),
        out_shape=jax.ShapeDtypeStruct((_N, _D), jnp.float32),
        compiler_params=pltpu.CompilerParams(
            dimension_semantics=("parallel",)),
    )(node_features, edge_features, mask2, *weights)


# speedup vs baseline: 1.6973x; 1.0511x over previous
import jax
import jax.numpy as jnp
from jax.experimental import pallas as pl
from jax.experimental.pallas import tpu as pltpu

_N, _K, _D, _L = 2048, 48, 128, 3
_BN = 512


def _g(x):
    return x * (1.0 + jax.lax.erf(x))


def _ln(x, w, b, eps=1e-5):
    mu = jnp.mean(x, axis=-1, keepdims=True)
    xc = x - mu
    var = jnp.mean(xc * xc, axis=-1, keepdims=True)
    return xc * jax.lax.rsqrt(var + eps) * w + b


def _decoder_kernel(nf_ref, edge_ref, mask_ref,
                    w0h_ref, w0n_ref, w0e_ref, b0_ref,
                    w1_ref, b1_ref, w2s_ref, b2s_ref,
                    ln1w_ref, ln1b_ref,
                    dw0_ref, db0_ref, dw1_ref, db1_ref,
                    ln2w_ref, ln2b_ref,
                    out_ref):
    nf = nf_ref[...]
    edge2 = edge_ref[...].reshape(_BN * _K, _D)
    h = nf
    e0 = jnp.dot(edge2, w0e_ref[0], preferred_element_type=jnp.float32)
    for l in range(_L):
        t0 = jnp.dot(h, w0h_ref[l], preferred_element_type=jnp.float32)
        t0 = t0 + jnp.dot(nf, w0n_ref[l], preferred_element_type=jnp.float32)
        t0 = t0 + b0_ref[l]
        x1 = _g(e0.reshape(_BN, _K, _D) + t0[:, None, :]).reshape(_BN * _K, _D)
        x2 = _g(jnp.dot(x1, w1_ref[l], preferred_element_type=jnp.float32)
                + b1_ref[l])
        if l + 1 < _L:
            e0 = jnp.dot(edge2, w0e_ref[l + 1],
                         preferred_element_type=jnp.float32)
        s = jnp.sum(x2.reshape(_BN, _K, _D), axis=1)
        agg = jnp.dot(s, w2s_ref[l], preferred_element_type=jnp.float32) + b2s_ref[l]
        h = _ln(h + agg, ln1w_ref[l], ln1b_ref[l])
        d1 = _g(jnp.dot(h, dw0_ref[l], preferred_element_type=jnp.float32)
                + db0_ref[l])
        d2 = jnp.dot(d1, dw1_ref[l], preferred_element_type=jnp.float32) \
            + db1_ref[l]
        h = _ln(h + d2, ln2w_ref[l], ln2b_ref[l])
    out_ref[...] = h * mask_ref[...]


def kernel(node_features, edge_features, mask, m_w0, m_b0, m_w1, m_b1, m_w2,
           m_b2, ln1_w, ln1_b, d_w0, d_b0, d_w1, d_b1, ln2_w, ln2_b):
    r = 0.7071067811865476
    tr = lambda w: jnp.transpose(w, (0, 2, 1))
    w0h = tr(m_w0[:, :, 0 * _D:1 * _D]) * r
    w0n = tr(m_w0[:, :, 1 * _D:2 * _D]) * r
    w0e = tr(m_w0[:, :, 3 * _D:4 * _D]) * r
    w1 = tr(m_w1) * 0.5
    w2s = tr(m_w2) * (r / 30.0)
    dw0 = tr(d_w0) * r
    dw1 = tr(d_w1) * r
    col = lambda b: b.reshape(_L, 1, _D)
    b0 = col(m_b0) * r
    b1 = col(m_b1) * r
    b2s = col(m_b2) * (_K / 30.0)
    mask2 = mask[:, None]

    full = lambda a: pl.BlockSpec(a.shape, lambda i: (0,) * a.ndim)
    weights = (w0h, w0n, w0e, b0, w1, b1, w2s, b2s,
               col(ln1_w), col(ln1_b), dw0, col(d_b0) * r, dw1, col(d_b1),
               col(ln2_w), col(ln2_b))
    return pl.pallas_call(
        _decoder_kernel,
        grid=(_N // _BN,),
        in_specs=[
            pl.BlockSpec((_BN, _D), lambda i: (i, 0)),
            pl.BlockSpec((_BN, _K, _D), lambda i: (i, 0, 0)),
            pl.BlockSpec((_BN, 1), lambda i: (i, 0)),
        ] + [full(w) for w in weights],
        out_specs=pl.BlockSpec((_BN, _D), lambda i: (i, 0)),
        out_shape=jax.ShapeDtypeStruct((_N, _D), jnp.float32),
        compiler_params=pltpu.CompilerParams(
            dimension_semantics=("parallel",)),
    )(node_features, edge_features, mask2, *weights)


# speedup vs baseline: 1.7822x; 1.0500x over previous
import jax
import jax.numpy as jnp
from jax.experimental import pallas as pl
from jax.experimental.pallas import tpu as pltpu

_N, _K, _D, _L = 2048, 48, 128, 3
_BN = 512


def _g(x):
    return x * (1.0 + jax.lax.erf(x))


def _ln(x, w, b, eps=1e-5):
    mu = jnp.mean(x, axis=-1, keepdims=True)
    xc = x - mu
    var = jnp.mean(xc * xc, axis=-1, keepdims=True)
    return xc * jax.lax.rsqrt(var + eps) * w + b


def _decoder_kernel(nf_ref, edge_ref, mask_ref,
                    w0h_ref, w0n_ref, w0e_ref, b0_ref,
                    w1_ref, b1_ref, w2s_ref, b2s_ref,
                    ln1w_ref, ln1b_ref,
                    dw0_ref, db0_ref, dw1_ref, db1_ref,
                    ln2w_ref, ln2b_ref,
                    out_ref):
    nf = nf_ref[...]
    edge2 = edge_ref[...].reshape(_BN * _K, _D).astype(jnp.bfloat16)
    h = nf
    e0 = jnp.dot(edge2, w0e_ref[0],
                 preferred_element_type=jnp.float32).astype(jnp.bfloat16)
    for l in range(_L):
        t0 = jnp.dot(h, w0h_ref[l], preferred_element_type=jnp.float32)
        t0 = t0 + jnp.dot(nf, w0n_ref[l], preferred_element_type=jnp.float32)
        t0 = (t0 + b0_ref[l]).astype(jnp.bfloat16)
        x1 = _g(e0.reshape(_BN, _K, _D) + t0[:, None, :]).reshape(_BN * _K, _D)
        x2 = _g(jnp.dot(x1, w1_ref[l],
                        preferred_element_type=jnp.float32).astype(jnp.bfloat16)
                + b1_ref[l])
        if l + 1 < _L:
            e0 = jnp.dot(edge2, w0e_ref[l + 1],
                         preferred_element_type=jnp.float32
                         ).astype(jnp.bfloat16)
        s = jnp.sum(x2.reshape(_BN, _K, _D), axis=1, dtype=jnp.float32)
        agg = jnp.dot(s, w2s_ref[l], preferred_element_type=jnp.float32) + b2s_ref[l]
        h = _ln(h + agg, ln1w_ref[l], ln1b_ref[l])
        d1 = _g(jnp.dot(h, dw0_ref[l], preferred_element_type=jnp.float32)
                + db0_ref[l])
        d2 = jnp.dot(d1, dw1_ref[l], preferred_element_type=jnp.float32) \
            + db1_ref[l]
        h = _ln(h + d2, ln2w_ref[l], ln2b_ref[l])
    out_ref[...] = h * mask_ref[...]


def kernel(node_features, edge_features, mask, m_w0, m_b0, m_w1, m_b1, m_w2,
           m_b2, ln1_w, ln1_b, d_w0, d_b0, d_w1, d_b1, ln2_w, ln2_b):
    r = 0.7071067811865476
    tr = lambda w: jnp.transpose(w, (0, 2, 1))
    w0h = tr(m_w0[:, :, 0 * _D:1 * _D]) * r
    w0n = tr(m_w0[:, :, 1 * _D:2 * _D]) * r
    w0e = (tr(m_w0[:, :, 3 * _D:4 * _D]) * r).astype(jnp.bfloat16)
    w1 = (tr(m_w1) * 0.5).astype(jnp.bfloat16)
    w2s = tr(m_w2) * (r / 30.0)
    dw0 = tr(d_w0) * r
    dw1 = tr(d_w1) * r
    col = lambda b: b.reshape(_L, 1, _D)
    b0 = col(m_b0) * r
    b1 = (col(m_b1) * r).astype(jnp.bfloat16)
    b2s = col(m_b2) * (_K / 30.0)
    mask2 = mask[:, None]

    full = lambda a: pl.BlockSpec(a.shape, lambda i: (0,) * a.ndim)
    weights = (w0h, w0n, w0e, b0, w1, b1, w2s, b2s,
               col(ln1_w), col(ln1_b), dw0, col(d_b0) * r, dw1, col(d_b1),
               col(ln2_w), col(ln2_b))
    return pl.pallas_call(
        _decoder_kernel,
        grid=(_N // _BN,),
        in_specs=[
            pl.BlockSpec((_BN, _D), lambda i: (i, 0)),
            pl.BlockSpec((_BN, _K, _D), lambda i: (i, 0, 0)),
            pl.BlockSpec((_BN, 1), lambda i: (i, 0)),
        ] + [full(w) for w in weights],
        out_specs=pl.BlockSpec((_BN, _D), lambda i: (i, 0)),
        out_shape=jax.ShapeDtypeStruct((_N, _D), jnp.float32),
        compiler_params=pltpu.CompilerParams(
            dimension_semantics=("parallel",)),
    )(node_features, edge_features, mask2, *weights)


# speedup vs baseline: 2.2629x; 1.2697x over previous
import jax
import jax.numpy as jnp
from jax.experimental import pallas as pl
from jax.experimental.pallas import tpu as pltpu

_N, _K, _D, _L = 2048, 48, 128, 3
_BN = 512
_R = 0.7071067811865476
_C11 = (((1,), (1,)), ((), ()))


def _g(x):
    return x * (1.0 + jax.lax.erf(x))


def _ln(x, w, b, eps=1e-5):
    mu = jnp.mean(x, axis=-1, keepdims=True)
    xc = x - mu
    var = jnp.mean(xc * xc, axis=-1, keepdims=True)
    return xc * jax.lax.rsqrt(var + eps) * w + b


def _dot(a, b):
    return jax.lax.dot_general(a, b, _C11,
                               preferred_element_type=jnp.float32)


def _decoder_kernel(nf_ref, edge_ref, mask_ref,
                    w0_ref, b0_ref, w1_ref, b1_ref, w2_ref, b2_ref,
                    ln1w_ref, ln1b_ref,
                    dw0_ref, db0_ref, dw1_ref, db1_ref,
                    ln2w_ref, ln2b_ref,
                    out_ref):
    nf = nf_ref[...]
    edge2 = edge_ref[...].reshape(_BN * _K, _D).astype(jnp.bfloat16)
    h = nf

    def w0e(l):
        return (w0_ref[l, :, 3 * _D:4 * _D] * _R).astype(jnp.bfloat16)

    e0 = _dot(edge2, w0e(0)).astype(jnp.bfloat16)
    for l in range(_L):
        t0 = _dot(h, w0_ref[l, :, 0:_D]) + _dot(nf, w0_ref[l, :, _D:2 * _D])
        t0 = ((t0 + b0_ref[l][None]) * _R).astype(jnp.bfloat16)
        x1 = _g(e0.reshape(_BN, _K, _D) + t0[:, None, :]).reshape(_BN * _K, _D)
        w1s = (w1_ref[l] * 0.5).astype(jnp.bfloat16)
        b1s = (b1_ref[l][None] * _R).astype(jnp.bfloat16)
        x2 = _g(_dot(x1, w1s).astype(jnp.bfloat16) + b1s)
        if l + 1 < _L:
            e0 = _dot(edge2, w0e(l + 1)).astype(jnp.bfloat16)
        s = jnp.sum(x2.reshape(_BN, _K, _D), axis=1, dtype=jnp.float32)
        agg = _dot(s, w2_ref[l]) * (_R / 30.0) + b2_ref[l][None] * (_K / 30.0)
        h = _ln(h + agg, ln1w_ref[l][None], ln1b_ref[l][None])
        d1 = _g((_dot(h, dw0_ref[l]) + db0_ref[l][None]) * _R)
        d2 = _dot(d1, dw1_ref[l]) * _R + db1_ref[l][None]
        h = _ln(h + d2, ln2w_ref[l][None], ln2b_ref[l][None])
    out_ref[...] = h * mask_ref[...]


def kernel(node_features, edge_features, mask, m_w0, m_b0, m_w1, m_b1, m_w2,
           m_b2, ln1_w, ln1_b, d_w0, d_b0, d_w1, d_b1, ln2_w, ln2_b):
    mask2 = mask[:, None]
    weights = (m_w0, m_b0, m_w1, m_b1, m_w2, m_b2, ln1_w, ln1_b,
               d_w0, d_b0, d_w1, d_b1, ln2_w, ln2_b)
    full = lambda a: pl.BlockSpec(a.shape, lambda i: (0,) * a.ndim)
    return pl.pallas_call(
        _decoder_kernel,
        grid=(_N // _BN,),
        in_specs=[
            pl.BlockSpec((_BN, _D), lambda i: (i, 0)),
            pl.BlockSpec((_BN, _K, _D), lambda i: (i, 0, 0)),
            pl.BlockSpec((_BN, 1), lambda i: (i, 0)),
        ] + [full(w) for w in weights],
        out_specs=pl.BlockSpec((_BN, _D), lambda i: (i, 0)),
        out_shape=jax.ShapeDtypeStruct((_N, _D), jnp.float32),
        compiler_params=pltpu.CompilerParams(
            dimension_semantics=("parallel",)),
    )(node_features, edge_features, mask2, *weights)
